# baseline (speedup 1.0000x reference)
import functools

import jax
import jax.numpy as jnp
from jax import lax
from jax.experimental import pallas as pl
from jax.experimental.pallas import tpu as pltpu
from jax.experimental.pallas import tpu_sc as plsc

N = 10000
E = 320000
D_STATE = 128
D_MSG = 64
H = 128
G3 = 3 * H
DE = 144
NC, NS, L = 2, 16, 16
NW = NC * NS
EPT = E // NW
EB = 80
NB = EPT // EB
ROWS_PT = 624
ROWS_REM = N - NS * ROWS_PT

_INTERPRET = False


def _encode_body(state_ref, msg_ref, w1_ref, b1_ref, w2_ref, b2_ref,
                 wih0t_ref, bih0_ref, gi0_ref):
    x = jnp.maximum(jnp.dot(state_ref[...], w1_ref[...],
                            preferred_element_type=jnp.float32) + b1_ref[...], 0.0)
    m = jnp.maximum(jnp.dot(msg_ref[...], w2_ref[...],
                            preferred_element_type=jnp.float32) + b2_ref[...], 0.0)
    x = x + m
    gi0_ref[...] = jnp.dot(x, wih0t_ref[...],
                           preferred_element_type=jnp.float32) + bih0_ref[...]


def _encode(state, message, W1, b1, W2, b2, Wih0T, bih0):
    TB = 2000
    grid = (N // TB,)
    return pl.pallas_call(
        _encode_body,
        grid=grid,
        in_specs=[
            pl.BlockSpec((TB, D_STATE), lambda i: (i, 0)),
            pl.BlockSpec((TB, D_MSG), lambda i: (i, 0)),
            pl.BlockSpec((D_STATE, H), lambda i: (0, 0)),
            pl.BlockSpec((1, H), lambda i: (0, 0)),
            pl.BlockSpec((D_MSG, H), lambda i: (0, 0)),
            pl.BlockSpec((1, H), lambda i: (0, 0)),
            pl.BlockSpec((H, G3), lambda i: (0, 0)),
            pl.BlockSpec((1, G3), lambda i: (0, 0)),
        ],
        out_specs=pl.BlockSpec((TB, G3), lambda i: (i, 0)),
        out_shape=jax.ShapeDtypeStruct((N, G3), jnp.float32),
        interpret=_INTERPRET,
    )(state, message, W1, b1.reshape(1, H), W2, b2.reshape(1, H),
      Wih0T, bih0.reshape(1, G3))


def _gru_body(gi0_ref, whh0t_ref, bhh0_ref, wih1t_ref, bih1_ref,
              whh1t_ref, bhh1_ref, y_ref, h0_s, h1_s, *, tb):
    @pl.when(pl.program_id(0) == 0)
    def _init():
        h0_s[...] = jnp.zeros_like(h0_s)
        h1_s[...] = jnp.zeros_like(h1_s)

    whh0t = whh0t_ref[...]
    wih1t = wih1t_ref[...]
    whh1t = whh1t_ref[...]
    bhh0 = bhh0_ref[...]
    bih1 = bih1_ref[...]
    bhh1 = bhh1_ref[...]

    def step(t, _):
        h0 = h0_s[...]
        h1 = h1_s[...]
        gi = jnp.broadcast_to(gi0_ref[pl.ds(t, 1), :], (8, G3))
        gh0 = jnp.dot(h0, whh0t, preferred_element_type=jnp.float32) + bhh0
        r0 = jax.nn.sigmoid(gi[:, 0:H] + gh0[:, 0:H])
        z0 = jax.nn.sigmoid(gi[:, H:2 * H] + gh0[:, H:2 * H])
        n0 = jnp.tanh(gi[:, 2 * H:] + r0 * gh0[:, 2 * H:])
        h0n = (1.0 - z0) * n0 + z0 * h0

        gi1 = jnp.dot(h0n, wih1t, preferred_element_type=jnp.float32) + bih1
        gh1 = jnp.dot(h1, whh1t, preferred_element_type=jnp.float32) + bhh1
        r1 = jax.nn.sigmoid(gi1[:, 0:H] + gh1[:, 0:H])
        z1 = jax.nn.sigmoid(gi1[:, H:2 * H] + gh1[:, H:2 * H])
        n1 = jnp.tanh(gi1[:, 2 * H:] + r1 * gh1[:, 2 * H:])
        h1n = (1.0 - z1) * n1 + z1 * h1

        h0_s[...] = h0n
        h1_s[...] = h1n
        y_ref[pl.ds(t, 1), :] = jnp.maximum(h1n[0:1, :], 0.0)
        return 0

    lax.fori_loop(0, tb, step, 0)


def _gru2(gi0, Whh0T, bhh0, Wih1T, bih1, Whh1T, bhh1):
    TB = 2000
    grid = (N // TB,)
    return pl.pallas_call(
        functools.partial(_gru_body, tb=TB),
        grid=grid,
        in_specs=[
            pl.BlockSpec((TB, G3), lambda i: (i, 0)),
            pl.BlockSpec((H, G3), lambda i: (0, 0)),
            pl.BlockSpec((1, G3), lambda i: (0, 0)),
            pl.BlockSpec((H, G3), lambda i: (0, 0)),
            pl.BlockSpec((1, G3), lambda i: (0, 0)),
            pl.BlockSpec((H, G3), lambda i: (0, 0)),
            pl.BlockSpec((1, G3), lambda i: (0, 0)),
        ],
        out_specs=pl.BlockSpec((TB, H), lambda i: (i, 0)),
        out_shape=jax.ShapeDtypeStruct((N, H), jnp.float32),
        scratch_shapes=[pltpu.VMEM((8, H), jnp.float32),
                        pltpu.VMEM((8, H), jnp.float32)],
        interpret=_INTERPRET,
    )(gi0, Whh0T, bhh0.reshape(1, G3), Wih1T, bih1.reshape(1, G3),
      Whh1T, bhh1.reshape(1, G3))


def _gat_edge_body(hext_hbm, asv_hbm, adv_hbm, src_hbm, dst_hbm, zeros_hbm,
                   out_hbm, asv_t, adv_t, src_blk, dst_blk, rows, p_buf,
                   acc, gsem):
    c = lax.axis_index("c")
    s = lax.axis_index("s")
    wid = c * NS + s

    pltpu.sync_copy(asv_hbm, asv_t)
    pltpu.sync_copy(adv_hbm, adv_t)
    off = s * ROWS_PT
    pltpu.sync_copy(zeros_hbm.at[pl.ds(off, ROWS_PT)], acc.at[pl.ds(off, ROWS_PT)])

    @pl.when(s == NS - 1)
    def _zrem():
        pltpu.sync_copy(zeros_hbm.at[pl.ds(NS * ROWS_PT, ROWS_REM)],
                        acc.at[pl.ds(NS * ROWS_PT, ROWS_REM)])

    plsc.subcore_barrier()

    def block(b, _):
        base = wid * EPT + b * EB
        pltpu.sync_copy(src_hbm.at[pl.ds(base, EB)], src_blk)
        pltpu.sync_copy(dst_hbm.at[pl.ds(base, EB)], dst_blk)
        pltpu.async_copy(hext_hbm.at[src_blk], rows, gsem).wait()
        for g in range(EB // L):
            sidx = src_blk[pl.ds(g * L, L)]
            didx = dst_blk[pl.ds(g * L, L)]
            e = plsc.load_gather(asv_t, [sidx]) + plsc.load_gather(adv_t, [didx])
            e = jnp.where(e >= 0.0, e, 0.2 * e)
            p_buf[pl.ds(g * L, L)] = jnp.exp(e)

        def scale(i, _):
            pb = plsc.load_gather(p_buf, [jnp.zeros((L,), jnp.int32) + i])
            for j in range(DE // L):
                sl = pl.ds(j * L, L)
                rows[i, sl] = rows[i, sl] * pb
            return 0

        lax.fori_loop(0, EB, scale, 0)
        pltpu.sync_copy(rows, acc.at[dst_blk], add=True)
        return 0

    lax.fori_loop(0, NB, block, 0)
    plsc.subcore_barrier()
    pltpu.sync_copy(acc.at[pl.ds(off, ROWS_PT)], out_hbm.at[c, pl.ds(off, ROWS_PT)])

    @pl.when(s == NS - 1)
    def _orem():
        pltpu.sync_copy(acc.at[pl.ds(NS * ROWS_PT, ROWS_REM)],
                        out_hbm.at[c, pl.ds(NS * ROWS_PT, ROWS_REM)])


def _gat_edge(hext, asv, adv, src, dst, zeros):
    mesh = plsc.VectorSubcoreMesh(core_axis_name="c", subcore_axis_name="s",
                                  num_cores=NC, num_subcores=NS)
    return pl.kernel(
        _gat_edge_body,
        out_type=jax.ShapeDtypeStruct((NC, N, DE), jnp.float32),
        mesh=mesh,
        scratch_types=[
            pltpu.VMEM((N,), jnp.float32),
            pltpu.VMEM((N,), jnp.float32),
            pltpu.VMEM((EB,), jnp.int32),
            pltpu.VMEM((EB,), jnp.int32),
            pltpu.VMEM((EB, DE), jnp.float32),
            pltpu.VMEM((EB,), jnp.float32),
            pltpu.VMEM_SHARED((N, DE), jnp.float32),
            pltpu.SemaphoreType.DMA,
        ],
        compiler_params=pltpu.CompilerParams(needs_layout_passes=False,
                                             use_tc_tiling_on_sc=False),
        interpret=_INTERPRET,
    )(hext, asv, adv, src, dst, zeros)


def _prep_body(x_ref, w_ref, as_ref, ad_ref, hext_ref, asv_ref, adv_ref):
    h = jnp.dot(x_ref[...], w_ref[...], preferred_element_type=jnp.float32)
    asv_ref[...] = jnp.dot(h, as_ref[...], preferred_element_type=jnp.float32)
    adv_ref[...] = jnp.dot(h, ad_ref[...], preferred_element_type=jnp.float32)
    tb = h.shape[0]
    lane = lax.broadcasted_iota(jnp.int32, (tb, DE - H), 1)
    pad = jnp.where(lane == 0, 1.0, 0.0)
    hext_ref[...] = jnp.concatenate([h, pad], axis=1)


def _gat_prep(x, W, a_s, a_d):
    TB = 2000
    grid = (N // TB,)
    return pl.pallas_call(
        _prep_body,
        grid=grid,
        in_specs=[
            pl.BlockSpec((TB, H), lambda i: (i, 0)),
            pl.BlockSpec((H, H), lambda i: (0, 0)),
            pl.BlockSpec((H, 1), lambda i: (0, 0)),
            pl.BlockSpec((H, 1), lambda i: (0, 0)),
        ],
        out_specs=[
            pl.BlockSpec((TB, DE), lambda i: (i, 0)),
            pl.BlockSpec((TB, 1), lambda i: (i, 0)),
            pl.BlockSpec((TB, 1), lambda i: (i, 0)),
        ],
        out_shape=[
            jax.ShapeDtypeStruct((N, DE), jnp.float32),
            jax.ShapeDtypeStruct((N, 1), jnp.float32),
            jax.ShapeDtypeStruct((N, 1), jnp.float32),
        ],
        interpret=_INTERPRET,
    )(x, W, a_s.reshape(H, 1), a_d.reshape(H, 1))


def _mid_body(g0_ref, g1_ref, w_ref, as_ref, ad_ref,
              hext_ref, asv_ref, adv_ref):
    g = g0_ref[...] + g1_ref[...]
    den = g[:, H:H + 1]
    x1 = jnp.maximum(g[:, 0:H] / (den + 1e-16), 0.0)
    h = jnp.dot(x1, w_ref[...], preferred_element_type=jnp.float32)
    asv_ref[...] = jnp.dot(h, as_ref[...], preferred_element_type=jnp.float32)
    adv_ref[...] = jnp.dot(h, ad_ref[...], preferred_element_type=jnp.float32)
    tb = h.shape[0]
    lane = lax.broadcasted_iota(jnp.int32, (tb, DE - H), 1)
    pad = jnp.where(lane == 0, 1.0, 0.0)
    hext_ref[...] = jnp.concatenate([h, pad], axis=1)


def _gat_mid(g0, g1, W, a_s, a_d):
    TB = 2000
    grid = (N // TB,)
    return pl.pallas_call(
        _mid_body,
        grid=grid,
        in_specs=[
            pl.BlockSpec((TB, DE), lambda i: (i, 0)),
            pl.BlockSpec((TB, DE), lambda i: (i, 0)),
            pl.BlockSpec((H, H), lambda i: (0, 0)),
            pl.BlockSpec((H, 1), lambda i: (0, 0)),
            pl.BlockSpec((H, 1), lambda i: (0, 0)),
        ],
        out_specs=[
            pl.BlockSpec((TB, DE), lambda i: (i, 0)),
            pl.BlockSpec((TB, 1), lambda i: (i, 0)),
            pl.BlockSpec((TB, 1), lambda i: (i, 0)),
        ],
        out_shape=[
            jax.ShapeDtypeStruct((N, DE), jnp.float32),
            jax.ShapeDtypeStruct((N, 1), jnp.float32),
            jax.ShapeDtypeStruct((N, 1), jnp.float32),
        ],
        interpret=_INTERPRET,
    )(g0, g1, W, a_s.reshape(H, 1), a_d.reshape(H, 1))


def _final_body(g0_ref, g1_ref, y_ref, wc_ref, bc_ref, wmuy_ref, wmug_ref,
                bmu_ref, wmsg_ref, bmsg_ref, comm_ref, msg_ref, mu_ref):
    g = g0_ref[...] + g1_ref[...]
    den = g[:, H:H + 1]
    xg = g[:, 0:H] / (den + 1e-16)
    y = y_ref[...]
    comm_ref[...] = jax.nn.sigmoid(
        jnp.dot(xg, wc_ref[...], preferred_element_type=jnp.float32) + bc_ref[...])
    mu_ref[...] = jnp.tanh(
        jnp.dot(y, wmuy_ref[...], preferred_element_type=jnp.float32)
        + jnp.dot(xg, wmug_ref[...], preferred_element_type=jnp.float32)
        + bmu_ref[...])
    msg_ref[...] = jnp.tanh(
        jnp.dot(xg, wmsg_ref[...], preferred_element_type=jnp.float32) + bmsg_ref[...])


def _finalize(g0, g1, y, Wc, bc, Wmu, bmu, Wmsg, bmsg):
    TB = 2000
    grid = (N // TB,)
    NA = Wmu.shape[1]
    MS = Wmsg.shape[1]
    return pl.pallas_call(
        _final_body,
        grid=grid,
        in_specs=[
            pl.BlockSpec((TB, DE), lambda i: (i, 0)),
            pl.BlockSpec((TB, DE), lambda i: (i, 0)),
            pl.BlockSpec((TB, H), lambda i: (i, 0)),
            pl.BlockSpec((H, 1), lambda i: (0, 0)),
            pl.BlockSpec((1, 1), lambda i: (0, 0)),
            pl.BlockSpec((H, NA), lambda i: (0, 0)),
            pl.BlockSpec((H, NA), lambda i: (0, 0)),
            pl.BlockSpec((1, NA), lambda i: (0, 0)),
            pl.BlockSpec((H, MS), lambda i: (0, 0)),
            pl.BlockSpec((1, MS), lambda i: (0, 0)),
        ],
        out_specs=[
            pl.BlockSpec((TB, 1), lambda i: (i, 0)),
            pl.BlockSpec((TB, MS), lambda i: (i, 0)),
            pl.BlockSpec((TB, NA), lambda i: (i, 0)),
        ],
        out_shape=[
            jax.ShapeDtypeStruct((N, 1), jnp.float32),
            jax.ShapeDtypeStruct((N, MS), jnp.float32),
            jax.ShapeDtypeStruct((N, NA), jnp.float32),
        ],
        interpret=_INTERPRET,
    )(g0, g1, y, Wc, bc.reshape(1, 1), Wmu[:H], Wmu[H:],
      bmu.reshape(1, NA), Wmsg, bmsg.reshape(1, MS))


def kernel(state, message, edge_index, W1, b1, W2, b2, Wih0, Whh0, bih0, bhh0,
           Wih1, Whh1, bih1, bhh1, Wg1, a1s, a1d, Wg2, a2s, a2d, Wc, bc,
           Wmu, bmu, Wmsg, bmsg):
    gi0 = _encode(state, message, W1, b1, W2, b2, Wih0.T, bih0)
    y = _gru2(gi0, Whh0.T, bhh0, Wih1.T, bih1, Whh1.T, bhh1)

    src = edge_index[0]
    dst = edge_index[1]
    zeros = jnp.zeros((N, DE), jnp.float32)

    hext1, asv1, adv1 = _gat_prep(y, Wg1, a1s, a1d)
    g1 = _gat_edge(hext1, asv1.reshape(N), adv1.reshape(N), src, dst, zeros)
    hext2, asv2, adv2 = _gat_mid(g1[0], g1[1], Wg2, a2s, a2d)
    g2 = _gat_edge(hext2, asv2.reshape(N), adv2.reshape(N), src, dst, zeros)

    comm, msg_out, mu = _finalize(g2[0], g2[1], y, Wc, bc, Wmu, bmu, Wmsg, bmsg)
    return (comm, msg_out, mu)


# speedup vs baseline: 1.3464x; 1.3464x over previous
import functools

import jax
import jax.numpy as jnp
from jax import lax
from jax.experimental import pallas as pl
from jax.experimental.pallas import tpu as pltpu
from jax.experimental.pallas import tpu_sc as plsc

N = 10000
E = 320000
D_STATE = 128
D_MSG = 64
H = 128
G3 = 3 * H
DE = 144
NC, NS, L = 2, 16, 16
NW = NC * NS
EPT = E // NW
EB = 80
NB = EPT // EB
ROWS_PT = 624
ROWS_REM = N - NS * ROWS_PT

_INTERPRET = False


def _encode_body(state_ref, msg_ref, w1_ref, b1_ref, w2_ref, b2_ref,
                 wih0t_ref, bih0_ref, gi0_ref):
    x = jnp.maximum(jnp.dot(state_ref[...], w1_ref[...],
                            preferred_element_type=jnp.float32) + b1_ref[...], 0.0)
    m = jnp.maximum(jnp.dot(msg_ref[...], w2_ref[...],
                            preferred_element_type=jnp.float32) + b2_ref[...], 0.0)
    x = x + m
    gi0_ref[...] = jnp.dot(x, wih0t_ref[...],
                           preferred_element_type=jnp.float32) + bih0_ref[...]


def _encode(state, message, W1, b1, W2, b2, Wih0T, bih0):
    TB = 2000
    grid = (N // TB,)
    return pl.pallas_call(
        _encode_body,
        grid=grid,
        in_specs=[
            pl.BlockSpec((TB, D_STATE), lambda i: (i, 0)),
            pl.BlockSpec((TB, D_MSG), lambda i: (i, 0)),
            pl.BlockSpec((D_STATE, H), lambda i: (0, 0)),
            pl.BlockSpec((1, H), lambda i: (0, 0)),
            pl.BlockSpec((D_MSG, H), lambda i: (0, 0)),
            pl.BlockSpec((1, H), lambda i: (0, 0)),
            pl.BlockSpec((H, G3), lambda i: (0, 0)),
            pl.BlockSpec((1, G3), lambda i: (0, 0)),
        ],
        out_specs=pl.BlockSpec((TB, G3), lambda i: (i, 0)),
        out_shape=jax.ShapeDtypeStruct((N, G3), jnp.float32),
        interpret=_INTERPRET,
    )(state, message, W1, b1.reshape(1, H), W2, b2.reshape(1, H),
      Wih0T, bih0.reshape(1, G3))


def _gru_gates(gi, gh, h):
    r = jax.nn.sigmoid(gi[:, 0:H] + gh[:, 0:H])
    z = jax.nn.sigmoid(gi[:, H:2 * H] + gh[:, H:2 * H])
    n = jnp.tanh(gi[:, 2 * H:] + r * gh[:, 2 * H:])
    return (1.0 - z) * n + z * h


def _gru_body(gi0_ref, wcat_ref, bcat_ref, wih1t_ref, bih1_ref, y_ref):
    wcat = wcat_ref[...]
    bcat = jnp.broadcast_to(bcat_ref[...], (8, 2 * G3))
    wih1t = wih1t_ref[...]
    bih1 = jnp.broadcast_to(bih1_ref[...], (8, G3))
    zeros8 = jnp.zeros((8, H), jnp.float32)

    gi00 = jnp.broadcast_to(gi0_ref[pl.ds(0, 1), :], (8, G3))
    h0 = _gru_gates(gi00, bcat[:, 0:G3], zeros8)

    def step(g, carry):
        h0, h1, y0p = carry
        u = jnp.concatenate([h0, h1], axis=1)
        gh01 = jnp.dot(u, wcat, preferred_element_type=jnp.float32) + bcat
        gi1 = jnp.dot(y0p, wih1t, preferred_element_type=jnp.float32) + bih1
        gi0t = jnp.broadcast_to(gi0_ref[pl.ds(g, 1), :], (8, G3))
        h0n = _gru_gates(gi0t, gh01[:, 0:G3], h0)
        h1n = _gru_gates(gi1, gh01[:, G3:], h1)
        y_ref[pl.ds(g - 1, 1), :] = jnp.maximum(h1n[0:1, :], 0.0)
        return (h0n, h1n, h0n)

    h0, h1, y0p = lax.fori_loop(1, N, step, (h0, zeros8, h0))
    gi1 = jnp.dot(y0p, wih1t, preferred_element_type=jnp.float32) + bih1
    gh1 = jnp.dot(h1, wcat_ref[H:2 * H, G3:],
                  preferred_element_type=jnp.float32) + bcat[:, G3:]
    h1n = _gru_gates(gi1, gh1, h1)
    y_ref[pl.ds(N - 1, 1), :] = jnp.maximum(h1n[0:1, :], 0.0)


def _gru2(gi0, Whh0T, bhh0, Wih1T, bih1, Whh1T, bhh1):
    z = jnp.zeros((H, G3), jnp.float32)
    wcat = jnp.concatenate([
        jnp.concatenate([Whh0T, z], axis=1),
        jnp.concatenate([z, Whh1T], axis=1)], axis=0)
    bcat = jnp.concatenate([bhh0, bhh1]).reshape(1, 2 * G3)
    return pl.pallas_call(
        _gru_body,
        grid=(1,),
        in_specs=[
            pl.BlockSpec((N, G3), lambda i: (0, 0)),
            pl.BlockSpec((2 * H, 2 * G3), lambda i: (0, 0)),
            pl.BlockSpec((1, 2 * G3), lambda i: (0, 0)),
            pl.BlockSpec((H, G3), lambda i: (0, 0)),
            pl.BlockSpec((1, G3), lambda i: (0, 0)),
        ],
        out_specs=pl.BlockSpec((N, H), lambda i: (0, 0)),
        out_shape=jax.ShapeDtypeStruct((N, H), jnp.float32),
        interpret=_INTERPRET,
    )(gi0, wcat, bcat, Wih1T, bih1.reshape(1, G3))


def _gat_edge_body(hext_hbm, asv_hbm, adv_hbm, src_hbm, dst_hbm, zeros_hbm,
                   out_hbm, asv_t, adv_t, src_blk, dst_blk, rows, p_buf,
                   acc, gsem):
    c = lax.axis_index("c")
    s = lax.axis_index("s")
    wid = c * NS + s

    pltpu.sync_copy(asv_hbm, asv_t)
    pltpu.sync_copy(adv_hbm, adv_t)
    off = s * ROWS_PT
    pltpu.sync_copy(zeros_hbm.at[pl.ds(off, ROWS_PT)], acc.at[pl.ds(off, ROWS_PT)])

    @pl.when(s == NS - 1)
    def _zrem():
        pltpu.sync_copy(zeros_hbm.at[pl.ds(NS * ROWS_PT, ROWS_REM)],
                        acc.at[pl.ds(NS * ROWS_PT, ROWS_REM)])

    plsc.subcore_barrier()

    def block(b, _):
        base = wid * EPT + b * EB
        pltpu.sync_copy(src_hbm.at[pl.ds(base, EB)], src_blk)
        pltpu.sync_copy(dst_hbm.at[pl.ds(base, EB)], dst_blk)
        pltpu.async_copy(hext_hbm.at[src_blk], rows, gsem).wait()
        for g in range(EB // L):
            sidx = src_blk[pl.ds(g * L, L)]
            didx = dst_blk[pl.ds(g * L, L)]
            e = plsc.load_gather(asv_t, [sidx]) + plsc.load_gather(adv_t, [didx])
            e = jnp.where(e >= 0.0, e, 0.2 * e)
            p_buf[pl.ds(g * L, L)] = jnp.exp(e)

        def scale(i, _):
            pb = plsc.load_gather(p_buf, [jnp.zeros((L,), jnp.int32) + i])
            for j in range(DE // L):
                sl = pl.ds(j * L, L)
                rows[i, sl] = rows[i, sl] * pb
            return 0

        lax.fori_loop(0, EB, scale, 0)
        pltpu.sync_copy(rows, acc.at[dst_blk], add=True)
        return 0

    lax.fori_loop(0, NB, block, 0)
    plsc.subcore_barrier()
    pltpu.sync_copy(acc.at[pl.ds(off, ROWS_PT)], out_hbm.at[c, pl.ds(off, ROWS_PT)])

    @pl.when(s == NS - 1)
    def _orem():
        pltpu.sync_copy(acc.at[pl.ds(NS * ROWS_PT, ROWS_REM)],
                        out_hbm.at[c, pl.ds(NS * ROWS_PT, ROWS_REM)])


def _gat_edge(hext, asv, adv, src, dst, zeros):
    mesh = plsc.VectorSubcoreMesh(core_axis_name="c", subcore_axis_name="s",
                                  num_cores=NC, num_subcores=NS)
    return pl.kernel(
        _gat_edge_body,
        out_type=jax.ShapeDtypeStruct((NC, N, DE), jnp.float32),
        mesh=mesh,
        scratch_types=[
            pltpu.VMEM((N,), jnp.float32),
            pltpu.VMEM((N,), jnp.float32),
            pltpu.VMEM((EB,), jnp.int32),
            pltpu.VMEM((EB,), jnp.int32),
            pltpu.VMEM((EB, DE), jnp.float32),
            pltpu.VMEM((EB,), jnp.float32),
            pltpu.VMEM_SHARED((N, DE), jnp.float32),
            pltpu.SemaphoreType.DMA,
        ],
        compiler_params=pltpu.CompilerParams(needs_layout_passes=False,
                                             use_tc_tiling_on_sc=False),
        interpret=_INTERPRET,
    )(hext, asv, adv, src, dst, zeros)


def _prep_body(x_ref, w_ref, as_ref, ad_ref, hext_ref, asv_ref, adv_ref):
    h = jnp.dot(x_ref[...], w_ref[...], preferred_element_type=jnp.float32)
    asv_ref[...] = jnp.dot(h, as_ref[...], preferred_element_type=jnp.float32)
    adv_ref[...] = jnp.dot(h, ad_ref[...], preferred_element_type=jnp.float32)
    tb = h.shape[0]
    lane = lax.broadcasted_iota(jnp.int32, (tb, DE - H), 1)
    pad = jnp.where(lane == 0, 1.0, 0.0)
    hext_ref[...] = jnp.concatenate([h, pad], axis=1)


def _gat_prep(x, W, a_s, a_d):
    TB = 2000
    grid = (N // TB,)
    return pl.pallas_call(
        _prep_body,
        grid=grid,
        in_specs=[
            pl.BlockSpec((TB, H), lambda i: (i, 0)),
            pl.BlockSpec((H, H), lambda i: (0, 0)),
            pl.BlockSpec((H, 1), lambda i: (0, 0)),
            pl.BlockSpec((H, 1), lambda i: (0, 0)),
        ],
        out_specs=[
            pl.BlockSpec((TB, DE), lambda i: (i, 0)),
            pl.BlockSpec((TB, 1), lambda i: (i, 0)),
            pl.BlockSpec((TB, 1), lambda i: (i, 0)),
        ],
        out_shape=[
            jax.ShapeDtypeStruct((N, DE), jnp.float32),
            jax.ShapeDtypeStruct((N, 1), jnp.float32),
            jax.ShapeDtypeStruct((N, 1), jnp.float32),
        ],
        interpret=_INTERPRET,
    )(x, W, a_s.reshape(H, 1), a_d.reshape(H, 1))


def _mid_body(g0_ref, g1_ref, w_ref, as_ref, ad_ref,
              hext_ref, asv_ref, adv_ref):
    g = g0_ref[...] + g1_ref[...]
    den = g[:, H:H + 1]
    x1 = jnp.maximum(g[:, 0:H] / (den + 1e-16), 0.0)
    h = jnp.dot(x1, w_ref[...], preferred_element_type=jnp.float32)
    asv_ref[...] = jnp.dot(h, as_ref[...], preferred_element_type=jnp.float32)
    adv_ref[...] = jnp.dot(h, ad_ref[...], preferred_element_type=jnp.float32)
    tb = h.shape[0]
    lane = lax.broadcasted_iota(jnp.int32, (tb, DE - H), 1)
    pad = jnp.where(lane == 0, 1.0, 0.0)
    hext_ref[...] = jnp.concatenate([h, pad], axis=1)


def _gat_mid(g0, g1, W, a_s, a_d):
    TB = 2000
    grid = (N // TB,)
    return pl.pallas_call(
        _mid_body,
        grid=grid,
        in_specs=[
            pl.BlockSpec((TB, DE), lambda i: (i, 0)),
            pl.BlockSpec((TB, DE), lambda i: (i, 0)),
            pl.BlockSpec((H, H), lambda i: (0, 0)),
            pl.BlockSpec((H, 1), lambda i: (0, 0)),
            pl.BlockSpec((H, 1), lambda i: (0, 0)),
        ],
        out_specs=[
            pl.BlockSpec((TB, DE), lambda i: (i, 0)),
            pl.BlockSpec((TB, 1), lambda i: (i, 0)),
            pl.BlockSpec((TB, 1), lambda i: (i, 0)),
        ],
        out_shape=[
            jax.ShapeDtypeStruct((N, DE), jnp.float32),
            jax.ShapeDtypeStruct((N, 1), jnp.float32),
            jax.ShapeDtypeStruct((N, 1), jnp.float32),
        ],
        interpret=_INTERPRET,
    )(g0, g1, W, a_s.reshape(H, 1), a_d.reshape(H, 1))


def _final_body(g0_ref, g1_ref, y_ref, wc_ref, bc_ref, wmuy_ref, wmug_ref,
                bmu_ref, wmsg_ref, bmsg_ref, comm_ref, msg_ref, mu_ref):
    g = g0_ref[...] + g1_ref[...]
    den = g[:, H:H + 1]
    xg = g[:, 0:H] / (den + 1e-16)
    y = y_ref[...]
    comm_ref[...] = jax.nn.sigmoid(
        jnp.dot(xg, wc_ref[...], preferred_element_type=jnp.float32) + bc_ref[...])
    mu_ref[...] = jnp.tanh(
        jnp.dot(y, wmuy_ref[...], preferred_element_type=jnp.float32)
        + jnp.dot(xg, wmug_ref[...], preferred_element_type=jnp.float32)
        + bmu_ref[...])
    msg_ref[...] = jnp.tanh(
        jnp.dot(xg, wmsg_ref[...], preferred_element_type=jnp.float32) + bmsg_ref[...])


def _finalize(g0, g1, y, Wc, bc, Wmu, bmu, Wmsg, bmsg):
    TB = 2000
    grid = (N // TB,)
    NA = Wmu.shape[1]
    MS = Wmsg.shape[1]
    return pl.pallas_call(
        _final_body,
        grid=grid,
        in_specs=[
            pl.BlockSpec((TB, DE), lambda i: (i, 0)),
            pl.BlockSpec((TB, DE), lambda i: (i, 0)),
            pl.BlockSpec((TB, H), lambda i: (i, 0)),
            pl.BlockSpec((H, 1), lambda i: (0, 0)),
            pl.BlockSpec((1, 1), lambda i: (0, 0)),
            pl.BlockSpec((H, NA), lambda i: (0, 0)),
            pl.BlockSpec((H, NA), lambda i: (0, 0)),
            pl.BlockSpec((1, NA), lambda i: (0, 0)),
            pl.BlockSpec((H, MS), lambda i: (0, 0)),
            pl.BlockSpec((1, MS), lambda i: (0, 0)),
        ],
        out_specs=[
            pl.BlockSpec((TB, 1), lambda i: (i, 0)),
            pl.BlockSpec((TB, MS), lambda i: (i, 0)),
            pl.BlockSpec((TB, NA), lambda i: (i, 0)),
        ],
        out_shape=[
            jax.ShapeDtypeStruct((N, 1), jnp.float32),
            jax.ShapeDtypeStruct((N, MS), jnp.float32),
            jax.ShapeDtypeStruct((N, NA), jnp.float32),
        ],
        interpret=_INTERPRET,
    )(g0, g1, y, Wc, bc.reshape(1, 1), Wmu[:H], Wmu[H:],
      bmu.reshape(1, NA), Wmsg, bmsg.reshape(1, MS))


def kernel(state, message, edge_index, W1, b1, W2, b2, Wih0, Whh0, bih0, bhh0,
           Wih1, Whh1, bih1, bhh1, Wg1, a1s, a1d, Wg2, a2s, a2d, Wc, bc,
           Wmu, bmu, Wmsg, bmsg):
    gi0 = _encode(state, message, W1, b1, W2, b2, Wih0.T, bih0)
    y = _gru2(gi0, Whh0.T, bhh0, Wih1.T, bih1, Whh1.T, bhh1)

    src = edge_index[0]
    dst = edge_index[1]
    zeros = jnp.zeros((N, DE), jnp.float32)

    hext1, asv1, adv1 = _gat_prep(y, Wg1, a1s, a1d)
    g1 = _gat_edge(hext1, asv1.reshape(N), adv1.reshape(N), src, dst, zeros)
    hext2, asv2, adv2 = _gat_mid(g1[0], g1[1], Wg2, a2s, a2d)
    g2 = _gat_edge(hext2, asv2.reshape(N), adv2.reshape(N), src, dst, zeros)

    comm, msg_out, mu = _finalize(g2[0], g2[1], y, Wc, bc, Wmu, bmu, Wmsg, bmsg)
    return (comm, msg_out, mu)
